# baseline (speedup 1.0000x reference)
import functools

import jax
import jax.numpy as jnp
from jax import lax
from jax.experimental import pallas as pl
from jax.experimental.pallas import tpu as pltpu
from jax.experimental.pallas import tpu_sc as plsc

M = 16
MM = M * M
NH = 16
B = 64
NIDX = MM * MM
TBL = (2 * M - 1) ** 2

_NC = 2
_NS = 16
_NW = _NC * _NS
_IPW = NIDX // _NW
_IROWS = MM // _NW


def _sc_gather_body(table_hbm, idx_hbm, out_hbm, table_v, idx_v, rows_v):
    wid = lax.axis_index("s") * _NC + lax.axis_index("c")
    pltpu.sync_copy(table_hbm, table_v)
    pltpu.sync_copy(idx_hbm.at[pl.ds(wid * _IPW, _IPW)], idx_v)

    def block(kb, carry):
        idx16 = idx_v[pl.ds(kb * 16, 16)]
        fidx = idx16 * NH
        i_loc = kb >> 4
        j0 = (kb & 15) * 16
        for h in range(NH):
            vals = plsc.load_gather(table_v, [fidx + h])
            rows_v[i_loc, h, pl.ds(j0, 16)] = vals
        return carry

    lax.fori_loop(0, _IPW // 16, block, 0, unroll=2)
    pltpu.sync_copy(rows_v, out_hbm.at[pl.ds(wid * _IROWS, _IROWS)])


@functools.cache
def _sc_gather():
    return pl.kernel(
        _sc_gather_body,
        out_type=jax.ShapeDtypeStruct((MM, NH, MM), jnp.float32),
        mesh=plsc.VectorSubcoreMesh(core_axis_name="c", subcore_axis_name="s"),
        scratch_types=[
            pltpu.VMEM((TBL * NH,), jnp.float32),
            pltpu.VMEM((_IPW,), jnp.int32),
            pltpu.VMEM((_IROWS, NH, MM), jnp.float32),
        ],
        compiler_params=pltpu.CompilerParams(
            needs_layout_passes=False, use_tc_tiling_on_sc=True
        ),
    )


def _add_body(xl_ref, xh_ref, bl_ref, bh_ref, o_ref):
    o_ref[:, : MM // 2] = xl_ref[:, 0] + bl_ref[0][None]
    o_ref[:, MM // 2 :] = xh_ref[:, 0] + bh_ref[0][None]


def kernel(x, bias_table, index):
    bias_t = _sc_gather()(bias_table.reshape(-1), index)
    xt = x.transpose(0, 1, 3, 2)
    xv = xt.reshape(B, 2, MM // 2, NH, MM)
    bv = bias_t.reshape(2, MM // 2, NH, MM)
    out_t = pl.pallas_call(
        _add_body,
        grid=(B // 2,),
        in_specs=[
            pl.BlockSpec((2, 1, MM // 2, NH, MM), lambda b: (b, 0, 0, 0, 0)),
            pl.BlockSpec((2, 1, MM // 2, NH, MM), lambda b: (b, 1, 0, 0, 0)),
            pl.BlockSpec((1, MM // 2, NH, MM), lambda b: (0, 0, 0, 0)),
            pl.BlockSpec((1, MM // 2, NH, MM), lambda b: (1, 0, 0, 0)),
        ],
        out_specs=pl.BlockSpec((2, MM, NH, MM), lambda b: (b, 0, 0, 0)),
        out_shape=jax.ShapeDtypeStruct((B, MM, NH, MM), jnp.float32),
    )(xv, xv, bv, bv)
    return out_t.transpose(0, 1, 3, 2)


# speedup vs baseline: 1.0051x; 1.0051x over previous
import functools

import jax
import jax.numpy as jnp
from jax import lax
from jax.experimental import pallas as pl
from jax.experimental.pallas import tpu as pltpu
from jax.experimental.pallas import tpu_sc as plsc

M = 16
MM = M * M
NH = 16
B = 64
NIDX = MM * MM
TBL = (2 * M - 1) ** 2

_NC = 2
_NS = 16
_NW = _NC * _NS
_IPW = NIDX // _NW
_IROWS = MM // _NW


def _sc_gather_body(table_hbm, idx_hbm, out_hbm, table_v, idx_v, rows_v, sem):
    wid = lax.axis_index("s") * _NC + lax.axis_index("c")
    ct = pltpu.async_copy(table_hbm, table_v, sem)
    ci = pltpu.async_copy(idx_hbm.at[pl.ds(wid * _IPW, _IPW)], idx_v, sem)
    ct.wait()
    ci.wait()

    def block(kb, carry):
        idx16 = idx_v[pl.ds(kb * 16, 16)]
        fidx = idx16 * NH
        i_loc = kb >> 4
        j0 = (kb & 15) * 16
        for h in range(NH):
            vals = plsc.load_gather(table_v, [fidx + h])
            rows_v[i_loc, h, pl.ds(j0, 16)] = vals
        return carry

    lax.fori_loop(0, _IPW // 16, block, 0, unroll=2)
    pltpu.sync_copy(rows_v, out_hbm.at[pl.ds(wid * _IROWS, _IROWS)])


@functools.cache
def _sc_gather():
    return pl.kernel(
        _sc_gather_body,
        out_type=jax.ShapeDtypeStruct((MM, NH, MM), jnp.float32),
        mesh=plsc.VectorSubcoreMesh(core_axis_name="c", subcore_axis_name="s"),
        scratch_types=[
            pltpu.VMEM((TBL * NH,), jnp.float32),
            pltpu.VMEM((_IPW,), jnp.int32),
            pltpu.VMEM((_IROWS, NH, MM), jnp.float32),
            pltpu.SemaphoreType.DMA,
        ],
        compiler_params=pltpu.CompilerParams(
            needs_layout_passes=False, use_tc_tiling_on_sc=True
        ),
    )


def _add_body(x_ref, b_ref, o_ref):
    o_ref[...] = x_ref[...] + b_ref[...][None]


def kernel(x, bias_table, index):
    bias_t = _sc_gather()(bias_table.reshape(-1), index)
    xt = x.transpose(0, 1, 3, 2)
    out_t = pl.pallas_call(
        _add_body,
        grid=(B // 2,),
        in_specs=[
            pl.BlockSpec((2, MM, NH, MM), lambda b: (b, 0, 0, 0)),
            pl.BlockSpec((MM, NH, MM), lambda b: (0, 0, 0)),
        ],
        out_specs=pl.BlockSpec((2, MM, NH, MM), lambda b: (b, 0, 0, 0)),
        out_shape=jax.ShapeDtypeStruct((B, MM, NH, MM), jnp.float32),
    )(xt, bias_t)
    return out_t.transpose(0, 1, 3, 2)
